# Initial kernel scaffold; baseline (speedup 1.0000x reference)
import functools

import jax
import jax.numpy as jnp
from jax import lax
from jax.experimental import pallas as pl
from jax.experimental.pallas import tpu as pltpu
from jax.experimental.pallas import tpu_sc as plsc

_N = 10000
_E = 320000
_DIN = 128

_NC = 2
_NS = 16
_NW = _NC * _NS
_EPW = _E // _NW
_CH = 80
_NCHUNK = _EPW // _CH
_RPS = _N // _NS
_DUMP = 125

_mesh = plsc.VectorSubcoreMesh(core_axis_name="c", subcore_axis_name="s")


@functools.partial(
    pl.kernel,
    out_type=jax.ShapeDtypeStruct((2, _NW, _N), jnp.float32),
    mesh=_mesh,
    scratch_types=[
        pltpu.VMEM((_EPW,), jnp.int32),
        pltpu.VMEM((_N,), jnp.float32),
        pltpu.VMEM((_N,), jnp.float32),
    ],
)
def _degree_kernel(src_hbm, dst_hbm, out_hbm, idx_v, hs_v, hd_v):
    cid = lax.axis_index("c")
    sid = lax.axis_index("s")
    wid = sid * _NC + cid
    zero16 = jnp.zeros((16,), jnp.float32)
    ones16 = jnp.ones((16,), jnp.float32)

    def zero_body(i, carry):
        hs_v[pl.ds(i * 16, 16)] = zero16
        hd_v[pl.ds(i * 16, 16)] = zero16
        return carry

    lax.fori_loop(0, _N // 16, zero_body, 0)

    pltpu.sync_copy(src_hbm.at[pl.ds(wid * _EPW, _EPW)], idx_v)

    def acc_src(i, carry):
        iv = idx_v[pl.ds(i * 16, 16)]
        plsc.addupdate_scatter(hs_v, [iv], ones16)
        return carry

    lax.fori_loop(0, _EPW // 16, acc_src, 0)

    pltpu.sync_copy(dst_hbm.at[pl.ds(wid * _EPW, _EPW)], idx_v)

    def acc_dst(i, carry):
        iv = idx_v[pl.ds(i * 16, 16)]
        plsc.addupdate_scatter(hd_v, [iv], ones16)
        return carry

    lax.fori_loop(0, _EPW // 16, acc_dst, 0)

    pltpu.sync_copy(hs_v, out_hbm.at[0, wid])
    pltpu.sync_copy(hd_v, out_hbm.at[1, wid])


def _make_agg_kernel(d):
    @functools.partial(
        pl.kernel,
        out_type=jax.ShapeDtypeStruct((_NC, _N, d), jnp.float32),
        mesh=_mesh,
        scratch_types=[
            pltpu.VMEM((_NCHUNK, _CH), jnp.int32),
            pltpu.VMEM((_NCHUNK, _CH), jnp.int32),
            pltpu.VMEM((_CH, d), jnp.float32),
            pltpu.VMEM((_DUMP, d), jnp.float32),
            pltpu.VMEM_SHARED((_N, d), jnp.float32),
            pltpu.SemaphoreType.DMA,
        ],
    )
    def agg_kernel(h_hbm, src_hbm, dst_hbm, out_hbm,
                   si_v, di_v, rows_v, zb_v, acc_sh, sem):
        cid = lax.axis_index("c")
        sid = lax.axis_index("s")
        wid = sid * _NC + cid
        zero16 = jnp.zeros((16,), jnp.float32)
        nsub = d // 16

        def zero_row(r, carry):
            def zero_col(j, carry2):
                zb_v[r, pl.ds(j * 16, 16)] = zero16
                return carry2
            return lax.fori_loop(0, nsub, zero_col, carry)

        lax.fori_loop(0, _DUMP, zero_row, 0)

        pltpu.sync_copy(src_hbm.at[wid], si_v)
        pltpu.sync_copy(dst_hbm.at[wid], di_v)
        for j in range(_RPS // _DUMP):
            pltpu.sync_copy(zb_v, acc_sh.at[pl.ds(sid * _RPS + j * _DUMP, _DUMP)])
        plsc.subcore_barrier()

        def body(i, carry):
            pltpu.async_copy(h_hbm.at[si_v.at[i]], rows_v, sem).wait()
            pltpu.sync_copy(rows_v, acc_sh.at[di_v.at[i]], add=True)
            return carry

        lax.fori_loop(0, _NCHUNK, body, 0)
        plsc.subcore_barrier()

        for j in range(_RPS // _DUMP):
            off = sid * _RPS + j * _DUMP
            pltpu.sync_copy(acc_sh.at[pl.ds(off, _DUMP)], zb_v)
            pltpu.sync_copy(zb_v, out_hbm.at[cid, pl.ds(off, _DUMP)])

    return agg_kernel


_agg64 = _make_agg_kernel(64)
_agg128 = _make_agg_kernel(128)


_RB = 2048
_GRID = (_N + _RB - 1) // _RB


def _norms(deg_blk):
    degsum = jnp.sum(deg_blk, axis=1)
    ns = lax.rsqrt(jnp.maximum(degsum[0], 1.0))
    nd = lax.rsqrt(jnp.maximum(degsum[1], 1.0))
    return ns, nd


def _first_body(x_ref, deg_ref, w_ref, o_ref):
    ns, _ = _norms(deg_ref[...])
    o_ref[...] = jnp.dot(x_ref[...] * ns[:, None], w_ref[...],
                         preferred_element_type=jnp.float32)


def _mid_body(agg_ref, deg_ref, b_ref, w_ref, o_ref):
    ns, nd = _norms(deg_ref[...])
    a = agg_ref[0] + agg_ref[1]
    x = jnp.maximum(a * nd[:, None] + b_ref[...], 0.0)
    o_ref[...] = jnp.dot(x * ns[:, None], w_ref[...],
                         preferred_element_type=jnp.float32)


def _last_body(agg_ref, deg_ref, b_ref, o_ref):
    _, nd = _norms(deg_ref[...])
    a = agg_ref[0] + agg_ref[1]
    o_ref[...] = jnp.maximum(a * nd[:, None] + b_ref[...], 0.0)


def _deg_spec():
    return pl.BlockSpec((2, _NW, _RB), lambda i: (0, 0, i))


def _tc_first(x, deg, w):
    dout = w.shape[1]
    return pl.pallas_call(
        _first_body,
        grid=(_GRID,),
        in_specs=[
            pl.BlockSpec((_RB, _DIN), lambda i: (i, 0)),
            _deg_spec(),
            pl.BlockSpec(w.shape, lambda i: (0, 0)),
        ],
        out_specs=pl.BlockSpec((_RB, dout), lambda i: (i, 0)),
        out_shape=jax.ShapeDtypeStruct((_N, dout), jnp.float32),
    )(x, deg, w)


def _tc_mid(agg, deg, b2d, w):
    din = agg.shape[2]
    dout = w.shape[1]
    return pl.pallas_call(
        _mid_body,
        grid=(_GRID,),
        in_specs=[
            pl.BlockSpec((_NC, _RB, din), lambda i: (0, i, 0)),
            _deg_spec(),
            pl.BlockSpec((1, din), lambda i: (0, 0)),
            pl.BlockSpec(w.shape, lambda i: (0, 0)),
        ],
        out_specs=pl.BlockSpec((_RB, dout), lambda i: (i, 0)),
        out_shape=jax.ShapeDtypeStruct((_N, dout), jnp.float32),
    )(agg, deg, b2d, w)


def _tc_last(agg, deg, b2d):
    din = agg.shape[2]
    return pl.pallas_call(
        _last_body,
        grid=(_GRID,),
        in_specs=[
            pl.BlockSpec((_NC, _RB, din), lambda i: (0, i, 0)),
            _deg_spec(),
            pl.BlockSpec((1, din), lambda i: (0, 0)),
        ],
        out_specs=pl.BlockSpec((_RB, din), lambda i: (i, 0)),
        out_shape=jax.ShapeDtypeStruct((_N, din), jnp.float32),
    )(agg, deg, b2d)


def kernel(features, edge_index, W1, b1, W2, b2, W3, b3, W4, b4):
    src = edge_index[0]
    dst = edge_index[1]
    src3 = src.reshape(_NW, _NCHUNK, _CH)
    dst3 = dst.reshape(_NW, _NCHUNK, _CH)

    deg = _degree_kernel(src, dst)

    h = _tc_first(features, deg, W1)
    agg = _agg64(h, src3, dst3)
    h = _tc_mid(agg, deg, b1.reshape(1, -1), W2)
    agg = _agg128(h, src3, dst3)
    h = _tc_mid(agg, deg, b2.reshape(1, -1), W3)
    agg = _agg128(h, src3, dst3)
    h = _tc_mid(agg, deg, b3.reshape(1, -1), W4)
    agg = _agg128(h, src3, dst3)
    f = _tc_last(agg, deg, b4.reshape(1, -1))
    return f


# speedup vs baseline: 4.1375x; 4.1375x over previous
import functools

import jax
import jax.numpy as jnp
from jax import lax
from jax.experimental import pallas as pl
from jax.experimental.pallas import tpu as pltpu
from jax.experimental.pallas import tpu_sc as plsc

_N = 10000
_E = 320000
_DIN = 128

_NC = 2
_NS = 16
_NW = _NC * _NS
_EPW = _E // _NW
_CH = 80
_NCHUNK = _EPW // _CH
_NP = 10240
_HALF = _NP // 2
_TRASH = _HALF
_ACCR = 5248
_ZD = _HALF // _NS
_BB = 128


@functools.cache
def _get_mesh():
    return plsc.VectorSubcoreMesh(core_axis_name="c", subcore_axis_name="s")


def _degree_body(src_hbm, dst_hbm, os_hbm, od_hbm, olo_hbm, ohi_hbm,
                 sv, dv, lo_v, hi_v, hs_v, hd_v):
    cid = lax.axis_index("c")
    sid = lax.axis_index("s")
    wid = sid * _NC + cid
    zero16 = jnp.zeros((16,), jnp.float32)
    ones16 = jnp.ones((16,), jnp.float32)
    half16 = jnp.full((16,), _HALF, jnp.int32)

    def zero_body(i, carry):
        hs_v[pl.ds(i * 16, 16)] = zero16
        hd_v[pl.ds(i * 16, 16)] = zero16
        return carry

    lax.fori_loop(0, _N // 16, zero_body, 0)

    pltpu.sync_copy(src_hbm.at[wid], sv)
    pltpu.sync_copy(dst_hbm.at[wid], dv)

    def acc_body(j, carry):
        r = j // 5
        c = (j % 5) * 16
        s16 = sv[r, pl.ds(c, 16)]
        d16 = dv[r, pl.ds(c, 16)]
        plsc.addupdate_scatter(hs_v, [s16], ones16)
        plsc.addupdate_scatter(hd_v, [d16], ones16)
        lo_v[r, pl.ds(c, 16)] = jnp.minimum(d16, half16)
        hi_v[r, pl.ds(c, 16)] = jnp.where(d16 >= half16, d16 - half16, half16)
        return carry

    lax.fori_loop(0, _EPW // 16, acc_body, 0)

    pltpu.sync_copy(hs_v, os_hbm.at[wid])
    pltpu.sync_copy(hd_v, od_hbm.at[wid])
    pltpu.sync_copy(lo_v, olo_hbm.at[wid])
    pltpu.sync_copy(hi_v, ohi_hbm.at[wid])


@functools.cache
def _get_degree_kernel():
    return functools.partial(
        pl.kernel,
        out_type=(jax.ShapeDtypeStruct((_NW, _N), jnp.float32),
                  jax.ShapeDtypeStruct((_NW, _N), jnp.float32),
                  jax.ShapeDtypeStruct((_NW, _NCHUNK, _CH), jnp.int32),
                  jax.ShapeDtypeStruct((_NW, _NCHUNK, _CH), jnp.int32)),
        mesh=_get_mesh(),
        compiler_params=pltpu.CompilerParams(needs_layout_passes=False),
        scratch_types=[
            pltpu.VMEM((_NCHUNK, _CH), jnp.int32),
            pltpu.VMEM((_NCHUNK, _CH), jnp.int32),
            pltpu.VMEM((_NCHUNK, _CH), jnp.int32),
            pltpu.VMEM((_NCHUNK, _CH), jnp.int32),
            pltpu.VMEM((_N,), jnp.float32),
            pltpu.VMEM((_N,), jnp.float32),
        ],
    )(_degree_body)


def _agg_body(h_hbm, src_hbm, dlo_hbm, dhi_hbm, out_hbm,
              si_v, dp_v, rows_v, zb_v, db_v, acc_sh, sem):
    cid = lax.axis_index("c")
    sid = lax.axis_index("s")
    wid = sid * _NC + cid
    zero16 = jnp.zeros((16,), jnp.float32)

    def zb_row(r, carry):
        def zb_col(j, carry2):
            zb_v[r, pl.ds(j * 16, 16)] = zero16
            return carry2
        return lax.fori_loop(0, _DIN // 16, zb_col, carry)

    lax.fori_loop(0, _BB, zb_row, 0)

    pltpu.sync_copy(src_hbm.at[wid], si_v)

    for p, didx_hbm in ((0, dlo_hbm), (1, dhi_hbm)):
        pltpu.sync_copy(didx_hbm.at[wid], dp_v)
        base = sid * _ZD
        for off, cnt in ((0, _BB), (_BB, _BB), (2 * _BB, _ZD - 2 * _BB)):
            pltpu.sync_copy(zb_v.at[pl.ds(0, cnt)], acc_sh.at[pl.ds(base + off, cnt)])
        plsc.subcore_barrier()

        def body(i, carry):
            pltpu.async_copy(h_hbm.at[si_v.at[i]], rows_v, sem).wait()
            pltpu.sync_copy(rows_v, acc_sh.at[dp_v.at[i]], add=True)
            return carry

        lax.fori_loop(0, _NCHUNK, body, 0)
        plsc.subcore_barrier()

        for off, cnt in ((0, _BB), (_BB, _BB), (2 * _BB, _ZD - 2 * _BB)):
            pltpu.sync_copy(acc_sh.at[pl.ds(base + off, cnt)], db_v.at[pl.ds(0, cnt)])
            pltpu.sync_copy(db_v.at[pl.ds(0, cnt)],
                            out_hbm.at[cid, pl.ds(p * _HALF + base + off, cnt)])
        plsc.subcore_barrier()


@functools.cache
def _get_agg_kernel():
    return functools.partial(
        pl.kernel,
        out_type=jax.ShapeDtypeStruct((_NC, _NP, _DIN), jnp.float32),
        mesh=_get_mesh(),
        compiler_params=pltpu.CompilerParams(needs_layout_passes=False),
        scratch_types=[
            pltpu.VMEM((_NCHUNK, _CH), jnp.int32),
            pltpu.VMEM((_NCHUNK, _CH), jnp.int32),
            pltpu.VMEM((_CH, _DIN), jnp.float32),
            pltpu.VMEM((_BB, _DIN), jnp.float32),
            pltpu.VMEM((_BB, _DIN), jnp.float32),
            pltpu.VMEM_SHARED((_ACCR, _DIN), jnp.float32),
            pltpu.SemaphoreType.DMA,
        ],
    )(_agg_body)


_RB = 2048
_GRID = (_N + _RB - 1) // _RB


def _first_body(x_ref, ds_ref, w_ref, o_ref):
    ns = lax.rsqrt(jnp.maximum(jnp.sum(ds_ref[...], axis=0), 1.0))
    o_ref[...] = jnp.dot(x_ref[...] * ns[:, None], w_ref[...],
                         preferred_element_type=jnp.float32)


def _mid_body(agg_ref, ds_ref, dd_ref, b_ref, w_ref, o_ref):
    ns = lax.rsqrt(jnp.maximum(jnp.sum(ds_ref[...], axis=0), 1.0))
    nd = lax.rsqrt(jnp.maximum(jnp.sum(dd_ref[...], axis=0), 1.0))
    a = agg_ref[0] + agg_ref[1]
    x = jnp.maximum(a * nd[:, None] + b_ref[...], 0.0)
    o_ref[...] = jnp.dot(x * ns[:, None], w_ref[...],
                         preferred_element_type=jnp.float32)


def _last_body(agg_ref, dd_ref, b_ref, o_ref):
    nd = lax.rsqrt(jnp.maximum(jnp.sum(dd_ref[...], axis=0), 1.0))
    a = agg_ref[0] + agg_ref[1]
    o_ref[...] = jnp.maximum(a * nd[:, None] + b_ref[...], 0.0)


def _deg_spec():
    return pl.BlockSpec((_NW, _RB), lambda i: (0, i))


def _tc_first(x, deg_s, w):
    dout = w.shape[1]
    return pl.pallas_call(
        _first_body,
        grid=(_GRID,),
        in_specs=[
            pl.BlockSpec((_RB, _DIN), lambda i: (i, 0)),
            _deg_spec(),
            pl.BlockSpec(w.shape, lambda i: (0, 0)),
        ],
        out_specs=pl.BlockSpec((_RB, dout), lambda i: (i, 0)),
        out_shape=jax.ShapeDtypeStruct((_N, dout), jnp.float32),
    )(x, deg_s, w)


def _tc_mid(agg, deg_s, deg_d, b2d, w):
    din = agg.shape[2]
    dout = w.shape[1]
    return pl.pallas_call(
        _mid_body,
        grid=(_GRID,),
        in_specs=[
            pl.BlockSpec((_NC, _RB, din), lambda i: (0, i, 0)),
            _deg_spec(),
            _deg_spec(),
            pl.BlockSpec((1, din), lambda i: (0, 0)),
            pl.BlockSpec(w.shape, lambda i: (0, 0)),
        ],
        out_specs=pl.BlockSpec((_RB, dout), lambda i: (i, 0)),
        out_shape=jax.ShapeDtypeStruct((_N, dout), jnp.float32),
    )(agg, deg_s, deg_d, b2d, w)


def _tc_last(agg, deg_d, b2d):
    din = agg.shape[2]
    return pl.pallas_call(
        _last_body,
        grid=(_GRID,),
        in_specs=[
            pl.BlockSpec((_NC, _RB, din), lambda i: (0, i, 0)),
            _deg_spec(),
            pl.BlockSpec((1, din), lambda i: (0, 0)),
        ],
        out_specs=pl.BlockSpec((_RB, din), lambda i: (i, 0)),
        out_shape=jax.ShapeDtypeStruct((_N, din), jnp.float32),
    )(agg, deg_d, b2d)


def kernel(features, edge_index, W1, b1, W2, b2, W3, b3, W4, b4):
    src3 = edge_index[0].reshape(_NW, _NCHUNK, _CH)
    dst3 = edge_index[1].reshape(_NW, _NCHUNK, _CH)

    deg_s, deg_d, dlo3, dhi3 = _get_degree_kernel()(src3, dst3)
    _agg = _get_agg_kernel()

    W1p = jnp.pad(W1, ((0, 0), (0, _DIN - W1.shape[1])))
    b1p = jnp.pad(b1, (0, _DIN - b1.shape[0]))
    W2p = jnp.pad(W2, ((0, _DIN - W2.shape[0]), (0, 0)))

    h = _tc_first(features, deg_s, W1p)
    agg = _agg(h, src3, dlo3, dhi3)
    h = _tc_mid(agg, deg_s, deg_d, b1p.reshape(1, -1), W2p)
    agg = _agg(h, src3, dlo3, dhi3)
    h = _tc_mid(agg, deg_s, deg_d, b2.reshape(1, -1), W3)
    agg = _agg(h, src3, dlo3, dhi3)
    h = _tc_mid(agg, deg_s, deg_d, b3.reshape(1, -1), W4)
    agg = _agg(h, src3, dlo3, dhi3)
    f = _tc_last(agg, deg_d, b4.reshape(1, -1))
    return f
